# Initial kernel scaffold; baseline (speedup 1.0000x reference)
import jax
import jax.numpy as jnp
from jax.experimental import pallas as pl


def kernel(score, target):
    raise NotImplementedError("write your pallas kernel here")


# speedup vs baseline: 19.3832x; 19.3832x over previous
import functools

import jax
import jax.numpy as jnp
from jax.experimental import pallas as pl
from jax.experimental.pallas import tpu as pltpu

_IGNORE_LABEL = -1
_THRESH = 0.7
_MIN_KEPT = 65535


def _stage1_body(score_ref, target_ref, loss_ref, pred_ref):
    s = score_ref[0]
    t = target_ref[0]
    m = jnp.max(s, axis=0)
    e = jnp.exp(s - m[None])
    denom = jnp.sum(e, axis=0)
    cls = jax.lax.broadcasted_iota(jnp.int32, s.shape, 0)
    s_t = jnp.sum(jnp.where(cls == t[None], s, 0.0), axis=0)
    loss = jnp.log(denom) - (s_t - m)
    loss_ref[0] = loss
    pred_ref[0] = jnp.exp(-loss)


def _stage2_body(pred_ref, loss_ref, out_ref, *, kth):
    pred = pred_ref[...]
    pb = jax.lax.bitcast_convert_type(pred, jnp.int32)

    def step(i, x):
        bit = 30 - i
        cand = x | jax.lax.shift_left(jnp.int32(1), bit)
        cnt = jnp.sum((pb < cand).astype(jnp.int32))
        return jnp.where(cnt <= kth, cand, x)

    xbits = jax.lax.fori_loop(0, 31, step, jnp.int32(0))
    vk = jax.lax.bitcast_convert_type(xbits, jnp.float32)
    th = jnp.maximum(vk, jnp.float32(_THRESH))
    keep = (pred < th).astype(jnp.float32)
    n_keep = jnp.maximum(jnp.sum(keep), 1.0)
    total = jnp.sum(loss_ref[...] * keep)
    out_ref[0, 0] = total / n_keep


@jax.jit
def kernel(score, target):
    B, C, H, W = score.shape
    Bh = 64
    grid = (B, H // Bh)

    loss, pred = pl.pallas_call(
        _stage1_body,
        grid=grid,
        in_specs=[
            pl.BlockSpec((1, C, Bh, W), lambda b, h: (b, 0, h, 0)),
            pl.BlockSpec((1, Bh, W), lambda b, h: (b, h, 0)),
        ],
        out_specs=[
            pl.BlockSpec((1, Bh, W), lambda b, h: (b, h, 0)),
            pl.BlockSpec((1, Bh, W), lambda b, h: (b, h, 0)),
        ],
        out_shape=[
            jax.ShapeDtypeStruct((B, H, W), jnp.float32),
            jax.ShapeDtypeStruct((B, H, W), jnp.float32),
        ],
    )(score, target)

    n = B * H * W
    rows = n // 1024
    pred2 = pred.reshape(rows, 1024)
    loss2 = loss.reshape(rows, 1024)
    kth = min(_MIN_KEPT, n - 1)

    out = pl.pallas_call(
        functools.partial(_stage2_body, kth=kth),
        out_specs=pl.BlockSpec(memory_space=pltpu.SMEM),
        out_shape=jax.ShapeDtypeStruct((1, 1), jnp.float32),
    )(pred2, loss2)
    return out[0, 0]
